# baseline (speedup 1.0000x reference)
import functools

import jax
import jax.numpy as jnp
from jax import lax
from jax.experimental import pallas as pl
from jax.experimental.pallas import tpu as pltpu
from jax.experimental.pallas import tpu_sc as plsc

BATCH = 4096
D_MODEL = 2048
N_FEATURES = 16384
K = 32

NC = 2
NS = 16
NW = NC * NS
RPW = BATCH // NW
NCH = 1024
NCV = NCH // 16
CAP = NCH + 16


def _encode_body(x_ref, bpre_ref, w_ref, benc_ref, pre_ref, cmax_ref):
    n = pl.program_id(1)
    xb = x_ref[...] - bpre_ref[...]
    acc = jnp.dot(xb, w_ref[...], preferred_element_type=jnp.float32)
    p = jnp.maximum(acc + benc_ref[...], 0.0)
    pre_ref[...] = p

    @pl.when(n % 16 == 0)
    def _init():
        cmax_ref[...] = p

    @pl.when(n % 16 != 0)
    def _acc():
        cmax_ref[...] = jnp.maximum(cmax_ref[...], p)


def _encode(x, b_pre, W_enc, b_enc, bm=1024, bn=512):
    m_blocks = BATCH // bm
    n_blocks = N_FEATURES // bn

    def c_of(n):
        return 2 * (n % 16) + n // 16

    return pl.pallas_call(
        _encode_body,
        grid=(m_blocks, n_blocks),
        in_specs=[
            pl.BlockSpec((bm, D_MODEL), lambda m, n: (m, 0)),
            pl.BlockSpec((1, D_MODEL), lambda m, n: (0, 0)),
            pl.BlockSpec((D_MODEL, bn), lambda m, n: (0, c_of(n))),
            pl.BlockSpec((1, bn), lambda m, n: (0, c_of(n))),
        ],
        out_specs=[
            pl.BlockSpec((bm, bn), lambda m, n: (m, c_of(n))),
            pl.BlockSpec((bm, bn), lambda m, n: (m, n // 16)),
        ],
        out_shape=[
            jax.ShapeDtypeStruct((BATCH, N_FEATURES), jnp.float32),
            jax.ShapeDtypeStruct((BATCH, NCH), jnp.float32),
        ],
    )(x, b_pre[None, :], W_enc, b_enc[None, :])


def _merge_chunk(Av, Ai, Bv, Bi, Cv, Ci):
    Cv, Ci = plsc.sort_key_val(Cv, Ci)
    Cr = lax.rev(Cv, (0,))
    Cir = lax.rev(Ci, (0,))
    m = Av >= Cr
    hv = jnp.where(m, Av, Cr)
    hi = jnp.where(m, Ai, Cir)
    lv = jnp.where(m, Cr, Av)
    li = jnp.where(m, Cir, Ai)
    Av, Ai = plsc.sort_key_val(hv, hi)
    lv, li = plsc.sort_key_val(lv, li)
    lr = lax.rev(lv, (0,))
    lir = lax.rev(li, (0,))
    m2 = Bv >= lr
    h2v = jnp.where(m2, Bv, lr)
    h2i = jnp.where(m2, Bi, lir)
    Bv, Bi = plsc.sort_key_val(h2v, h2i)
    return Av, Ai, Bv, Bi


_ABL_MERGE = 1


def _splat0(v):
    return lax.gather(
        v, jnp.zeros((16, 1), jnp.int32),
        lax.GatherDimensionNumbers(
            offset_dims=(), collapsed_slice_dims=(0,), start_index_map=(0,)),
        (1,), mode=lax.GatherScatterMode.PROMISE_IN_BOUNDS)


def _sc_body(pre_hbm, cmax_hbm, h_hbm,
             pre0, pre1, h0, h1, cm0, cm1, cids0, cids1,
             piA0, piB0, piA1, piB1, sem_pre, sem_cm, sem_h):
    wid = lax.axis_index("s") * NC + lax.axis_index("c")
    base = wid * RPW
    iota = lax.iota(jnp.int32, 16)
    zeros16 = jnp.zeros((16,), jnp.float32)
    neg1 = jnp.full((16,), -1.0, jnp.float32)

    def zstep(j, c):
        h0[pl.ds(j * 16, 16)] = zeros16
        h1[pl.ds(j * 16, 16)] = zeros16
        return c
    lax.fori_loop(0, N_FEATURES // 16, zstep, 0)
    piA0[...] = iota
    piB0[...] = iota + 16
    piA1[...] = iota
    piB1[...] = iota + 16

    pltpu.async_copy(pre_hbm.at[base], pre0.at[:, pl.ds(0, NCH)], sem_pre)
    pltpu.async_copy(cmax_hbm.at[base], cm0, sem_cm)
    pltpu.async_copy(pre_hbm.at[base + 1], pre1.at[:, pl.ds(0, NCH)], sem_pre)
    pltpu.async_copy(cmax_hbm.at[base + 1], cm1, sem_cm)

    def outer(r2, carry):
        row0 = base + 2 * r2
        row1 = row0 + 1
        pltpu.make_async_copy(pre_hbm.at[row0], pre0.at[:, pl.ds(0, NCH)],
                              sem_pre).wait()
        pltpu.make_async_copy(cmax_hbm.at[row0], cm0, sem_cm).wait()
        pltpu.make_async_copy(pre_hbm.at[row1], pre1.at[:, pl.ds(0, NCH)],
                              sem_pre).wait()
        pltpu.make_async_copy(cmax_hbm.at[row1], cm1, sem_cm).wait()

        def t0step(j, LL):
            L1a, L2a, L1b, L2b = LL
            va = cm0[pl.ds(j * 16, 16)]
            vb = cm1[pl.ds(j * 16, 16)]
            ma = va > L1a
            mb = vb > L1b
            L2a = jnp.where(ma, L1a, jnp.maximum(L2a, va))
            L1a = jnp.where(ma, va, L1a)
            L2b = jnp.where(mb, L1b, jnp.maximum(L2b, vb))
            L1b = jnp.where(mb, vb, L1b)
            return (L1a, L2a, L1b, L2b)
        _, L2a, _, L2b = lax.fori_loop(0, NCV, t0step,
                                       (neg1, neg1, neg1, neg1))
        sL2a, _ = plsc.sort_key_val(L2a, iota)
        sL2b, _ = plsc.sort_key_val(L2b, iota)
        t0a = _splat0(sL2a)
        t0b = _splat0(sL2b)

        def cstep(j, cnts):
            cnta, cntb = cnts
            ids = iota + j * 16
            va = cm0[pl.ds(j * 16, 16)]
            vb = cm1[pl.ds(j * 16, 16)]
            ma = va >= t0a
            mb = vb >= t0b
            posa = cnta + plsc.cumsum(ma.astype(jnp.int32)) - 1
            posb = cntb + plsc.cumsum(mb.astype(jnp.int32)) - 1
            plsc.store_scatter(cids0, [posa], ids, mask=ma)
            plsc.store_scatter(cids1, [posb], ids, mask=mb)
            return (cnta + plsc.all_reduce_population_count(ma),
                    cntb + plsc.all_reduce_population_count(mb))
        cnta, cntb = lax.fori_loop(
            0, NCV, cstep,
            (jnp.zeros((16,), jnp.int32), jnp.zeros((16,), jnp.int32)))
        cntmax = jnp.maximum(cnta, cntb)

        def mcond(state):
            jv = state[0]
            return jnp.any(jv < cntmax * _ABL_MERGE)

        def mbody(state):
            jv, Aa, Iaa, Ba, Iba, Ab, Iab, Bb, Ibb = state
            cida = plsc.load_gather(cids0, [jv])
            cidb = plsc.load_gather(cids1, [jv])
            cida = jnp.where(jv < cnta, cida, 0)
            cidb = jnp.where(jv < cntb, cidb, 0)
            eidxa = cida + NCH * iota
            eidxb = cidb + NCH * iota
            Cva = plsc.load_gather(pre0, [iota, cida])
            Cvb = plsc.load_gather(pre1, [iota, cidb])
            Cva = jnp.where(jv < cnta, Cva, neg1)
            Cvb = jnp.where(jv < cntb, Cvb, neg1)
            Aa, Iaa, Ba, Iba = _merge_chunk(Aa, Iaa, Ba, Iba, Cva, eidxa)
            Ab, Iab, Bb, Ibb = _merge_chunk(Ab, Iab, Bb, Ibb, Cvb, eidxb)
            return (jv + 1, Aa, Iaa, Ba, Iba, Ab, Iab, Bb, Ibb)
        init = (jnp.zeros((16,), jnp.int32),
                zeros16, iota, zeros16, iota + 16,
                zeros16, iota, zeros16, iota + 16)
        (_, Aa, Iaa, Ba, Iba, Ab, Iab, Bb, Ibb) = lax.while_loop(
            mcond, mbody, init)

        @pl.when(r2 > 0)
        def _wait_h():
            pltpu.make_async_copy(h0, h_hbm.at[row0], sem_h).wait()
            pltpu.make_async_copy(h1, h_hbm.at[row1], sem_h).wait()
        plsc.store_scatter(h0, [piA0[...]], zeros16)
        plsc.store_scatter(h0, [piB0[...]], zeros16)
        plsc.store_scatter(h1, [piA1[...]], zeros16)
        plsc.store_scatter(h1, [piB1[...]], zeros16)
        plsc.store_scatter(h0, [Iaa], Aa)
        plsc.store_scatter(h0, [Iba], Ba)
        plsc.store_scatter(h1, [Iab], Ab)
        plsc.store_scatter(h1, [Ibb], Bb)
        piA0[...] = Iaa
        piB0[...] = Iba
        piA1[...] = Iab
        piB1[...] = Ibb
        pltpu.async_copy(h0, h_hbm.at[row0], sem_h)
        pltpu.async_copy(h1, h_hbm.at[row1], sem_h)

        @pl.when(r2 < RPW // 2 - 1)
        def _prefetch():
            pltpu.async_copy(pre_hbm.at[row0 + 2],
                             pre0.at[:, pl.ds(0, NCH)], sem_pre)
            pltpu.async_copy(cmax_hbm.at[row0 + 2], cm0, sem_cm)
            pltpu.async_copy(pre_hbm.at[row1 + 2],
                             pre1.at[:, pl.ds(0, NCH)], sem_pre)
            pltpu.async_copy(cmax_hbm.at[row1 + 2], cm1, sem_cm)
        return carry

    lax.fori_loop(0, RPW // 2, outer, jnp.int32(0))

    pltpu.make_async_copy(h0, h_hbm.at[base], sem_h).wait()
    pltpu.make_async_copy(h1, h_hbm.at[base], sem_h).wait()


def _sc_topk(pre, cmax):
    mesh = plsc.VectorSubcoreMesh(core_axis_name="c", subcore_axis_name="s")
    f = pl.kernel(
        _sc_body,
        out_type=jax.ShapeDtypeStruct((BATCH, N_FEATURES), jnp.float32),
        mesh=mesh,
        compiler_params=pltpu.CompilerParams(needs_layout_passes=False),
        scratch_types=[
            pltpu.VMEM((16, NCH + 1), jnp.float32),
            pltpu.VMEM((16, NCH + 1), jnp.float32),
            pltpu.VMEM((N_FEATURES,), jnp.float32),
            pltpu.VMEM((N_FEATURES,), jnp.float32),
            pltpu.VMEM((NCH,), jnp.float32),
            pltpu.VMEM((NCH,), jnp.float32),
            pltpu.VMEM((CAP,), jnp.int32),
            pltpu.VMEM((CAP,), jnp.int32),
            pltpu.VMEM((16,), jnp.int32),
            pltpu.VMEM((16,), jnp.int32),
            pltpu.VMEM((16,), jnp.int32),
            pltpu.VMEM((16,), jnp.int32),
            pltpu.SemaphoreType.DMA,
            pltpu.SemaphoreType.DMA,
            pltpu.SemaphoreType.DMA,
        ],
    )
    return f(pre, cmax)


def _decode_body(h_ref, w_ref, bdec_ref, out_ref):
    k = pl.program_id(1)
    acc = jnp.dot(h_ref[...], w_ref[...], preferred_element_type=jnp.float32)

    @pl.when(k == 0)
    def _init():
        out_ref[...] = acc + bdec_ref[...]

    @pl.when(k != 0)
    def _acc():
        out_ref[...] += acc


def _decode(h, W_dec, b_dec, bm=1024, bk=1024):
    m_blocks = BATCH // bm
    k_blocks = N_FEATURES // bk
    return pl.pallas_call(
        _decode_body,
        grid=(m_blocks, k_blocks),
        in_specs=[
            pl.BlockSpec((bm, bk), lambda m, k: (m, k)),
            pl.BlockSpec((bk, D_MODEL), lambda m, k: (k, 0)),
            pl.BlockSpec((1, D_MODEL), lambda m, k: (0, 0)),
        ],
        out_specs=pl.BlockSpec((bm, D_MODEL), lambda m, k: (m, 0)),
        out_shape=jax.ShapeDtypeStruct((BATCH, D_MODEL), jnp.float32),
    )(h, W_dec, b_dec[None, :])


def kernel(x, b_pre, W_enc, b_enc, W_dec, b_dec):
    pre, cmax = _encode(x, b_pre, W_enc, b_enc)
    h = _sc_topk(pre.reshape(BATCH, 16, NCH), cmax)
    x_hat = _decode(h, W_dec, b_dec)
    return (h, x_hat)


# speedup vs baseline: 1.1267x; 1.1267x over previous
import functools

import jax
import jax.numpy as jnp
from jax import lax
from jax.experimental import pallas as pl
from jax.experimental.pallas import tpu as pltpu
from jax.experimental.pallas import tpu_sc as plsc

BATCH = 4096
D_MODEL = 2048
N_FEATURES = 16384
K = 32

NC = 2
NS = 16
NW = NC * NS
RPW = BATCH // NW
NCH = 1024
NCV = NCH // 16
CAP = NCH + 16


def _encode_body(x_ref, bpre_ref, w_ref, benc_ref, pre_ref, cmax_ref):
    n = pl.program_id(1)
    xb = x_ref[...] - bpre_ref[...]
    acc = jnp.dot(xb, w_ref[...], preferred_element_type=jnp.float32)
    p = jnp.maximum(acc + benc_ref[...], 0.0)
    pre_ref[...] = p

    @pl.when(n % 16 == 0)
    def _init():
        cmax_ref[...] = p

    @pl.when(n % 16 != 0)
    def _acc():
        cmax_ref[...] = jnp.maximum(cmax_ref[...], p)


def _encode(x, b_pre, W_enc, b_enc, bm=1024, bn=512):
    m_blocks = BATCH // bm
    n_blocks = N_FEATURES // bn

    def c_of(n):
        return 2 * (n % 16) + n // 16

    return pl.pallas_call(
        _encode_body,
        grid=(m_blocks, n_blocks),
        in_specs=[
            pl.BlockSpec((bm, D_MODEL), lambda m, n: (m, 0)),
            pl.BlockSpec((1, D_MODEL), lambda m, n: (0, 0)),
            pl.BlockSpec((D_MODEL, bn), lambda m, n: (0, c_of(n))),
            pl.BlockSpec((1, bn), lambda m, n: (0, c_of(n))),
        ],
        out_specs=[
            pl.BlockSpec((bm, bn), lambda m, n: (m, c_of(n))),
            pl.BlockSpec((bm, bn), lambda m, n: (m, n // 16)),
        ],
        out_shape=[
            jax.ShapeDtypeStruct((BATCH, N_FEATURES), jnp.float32),
            jax.ShapeDtypeStruct((BATCH, NCH), jnp.float32),
        ],
    )(x, b_pre[None, :], W_enc, b_enc[None, :])


def _merge_chunk(Av, Ai, Bv, Bi, Cv, Ci):
    Cv, Ci = plsc.sort_key_val(Cv, Ci)
    Cr = lax.rev(Cv, (0,))
    Cir = lax.rev(Ci, (0,))
    m = Av >= Cr
    hv = jnp.where(m, Av, Cr)
    hi = jnp.where(m, Ai, Cir)
    lv = jnp.where(m, Cr, Av)
    li = jnp.where(m, Cir, Ai)
    Av, Ai = plsc.sort_key_val(hv, hi)
    lv, li = plsc.sort_key_val(lv, li)
    lr = lax.rev(lv, (0,))
    lir = lax.rev(li, (0,))
    m2 = Bv >= lr
    h2v = jnp.where(m2, Bv, lr)
    h2i = jnp.where(m2, Bi, lir)
    Bv, Bi = plsc.sort_key_val(h2v, h2i)
    return Av, Ai, Bv, Bi


_ABL_MERGE = 1


def _splat0(v):
    return lax.gather(
        v, jnp.zeros((16, 1), jnp.int32),
        lax.GatherDimensionNumbers(
            offset_dims=(), collapsed_slice_dims=(0,), start_index_map=(0,)),
        (1,), mode=lax.GatherScatterMode.PROMISE_IN_BOUNDS)


def _sc_body(pre_hbm, cmax_hbm, h_hbm,
             pre0, pre1, h0, h1, cm0, cm1, cids0, cids1,
             piA0, piB0, piA1, piB1, sem_pre, sem_cm, sem_h):
    wid = lax.axis_index("s") * NC + lax.axis_index("c")
    base = wid * RPW
    iota = lax.iota(jnp.int32, 16)
    zeros16 = jnp.zeros((16,), jnp.float32)
    neg1 = jnp.full((16,), -1.0, jnp.float32)

    def zstep(j, c):
        h0[pl.ds(j * 16, 16)] = zeros16
        h1[pl.ds(j * 16, 16)] = zeros16
        return c
    lax.fori_loop(0, N_FEATURES // 16, zstep, 0)
    piA0[...] = iota
    piB0[...] = iota + 16
    piA1[...] = iota
    piB1[...] = iota + 16

    def _fetch_pre(row, dst):
        for l in range(16):
            pltpu.async_copy(pre_hbm.at[row, pl.ds(l * NCH, NCH)],
                             dst.at[l, pl.ds(0, NCH)], sem_pre)

    _fetch_pre(base, pre0)
    pltpu.async_copy(cmax_hbm.at[base], cm0, sem_cm)
    _fetch_pre(base + 1, pre1)
    pltpu.async_copy(cmax_hbm.at[base + 1], cm1, sem_cm)

    def outer(r2, carry):
        row0 = base + 2 * r2
        row1 = row0 + 1
        for l in range(16):
            pltpu.make_async_copy(pre_hbm.at[row0, pl.ds(l * NCH, NCH)],
                                  pre0.at[l, pl.ds(0, NCH)], sem_pre).wait()
            pltpu.make_async_copy(pre_hbm.at[row1, pl.ds(l * NCH, NCH)],
                                  pre1.at[l, pl.ds(0, NCH)], sem_pre).wait()
        pltpu.make_async_copy(cmax_hbm.at[row0], cm0, sem_cm).wait()
        pltpu.make_async_copy(cmax_hbm.at[row1], cm1, sem_cm).wait()

        def t0step(j, LL):
            L1a, L2a, L1b, L2b = LL
            va = cm0[pl.ds(j * 16, 16)]
            vb = cm1[pl.ds(j * 16, 16)]
            ma = va > L1a
            mb = vb > L1b
            L2a = jnp.where(ma, L1a, jnp.maximum(L2a, va))
            L1a = jnp.where(ma, va, L1a)
            L2b = jnp.where(mb, L1b, jnp.maximum(L2b, vb))
            L1b = jnp.where(mb, vb, L1b)
            return (L1a, L2a, L1b, L2b)
        _, L2a, _, L2b = lax.fori_loop(0, NCV, t0step,
                                       (neg1, neg1, neg1, neg1))
        sL2a, _ = plsc.sort_key_val(L2a, iota)
        sL2b, _ = plsc.sort_key_val(L2b, iota)
        t0a = _splat0(sL2a)
        t0b = _splat0(sL2b)

        def cstep(j, cnts):
            cnta, cntb = cnts
            ids = iota + j * 16
            va = cm0[pl.ds(j * 16, 16)]
            vb = cm1[pl.ds(j * 16, 16)]
            ma = va >= t0a
            mb = vb >= t0b
            posa = cnta + plsc.cumsum(ma.astype(jnp.int32)) - 1
            posb = cntb + plsc.cumsum(mb.astype(jnp.int32)) - 1
            plsc.store_scatter(cids0, [posa], ids, mask=ma)
            plsc.store_scatter(cids1, [posb], ids, mask=mb)
            return (cnta + plsc.all_reduce_population_count(ma),
                    cntb + plsc.all_reduce_population_count(mb))
        cnta, cntb = lax.fori_loop(
            0, NCV, cstep,
            (jnp.zeros((16,), jnp.int32), jnp.zeros((16,), jnp.int32)))
        cntmax = jnp.maximum(cnta, cntb)

        def mcond(state):
            jv = state[0]
            return jnp.any(jv < cntmax * _ABL_MERGE)

        def mbody(state):
            jv, Aa, Iaa, Ba, Iba, Ab, Iab, Bb, Ibb = state
            cida = plsc.load_gather(cids0, [jv])
            cidb = plsc.load_gather(cids1, [jv])
            cida = jnp.where(jv < cnta, cida, 0)
            cidb = jnp.where(jv < cntb, cidb, 0)
            eidxa = cida + NCH * iota
            eidxb = cidb + NCH * iota
            Cva = plsc.load_gather(pre0, [iota, cida])
            Cvb = plsc.load_gather(pre1, [iota, cidb])
            Cva = jnp.where(jv < cnta, Cva, neg1)
            Cvb = jnp.where(jv < cntb, Cvb, neg1)
            Aa, Iaa, Ba, Iba = _merge_chunk(Aa, Iaa, Ba, Iba, Cva, eidxa)
            Ab, Iab, Bb, Ibb = _merge_chunk(Ab, Iab, Bb, Ibb, Cvb, eidxb)
            return (jv + 1, Aa, Iaa, Ba, Iba, Ab, Iab, Bb, Ibb)
        init = (jnp.zeros((16,), jnp.int32),
                zeros16, iota, zeros16, iota + 16,
                zeros16, iota, zeros16, iota + 16)
        (_, Aa, Iaa, Ba, Iba, Ab, Iab, Bb, Ibb) = lax.while_loop(
            mcond, mbody, init)

        @pl.when(r2 > 0)
        def _wait_h():
            pltpu.make_async_copy(h0, h_hbm.at[row0], sem_h).wait()
            pltpu.make_async_copy(h1, h_hbm.at[row1], sem_h).wait()
        plsc.store_scatter(h0, [piA0[...]], zeros16)
        plsc.store_scatter(h0, [piB0[...]], zeros16)
        plsc.store_scatter(h1, [piA1[...]], zeros16)
        plsc.store_scatter(h1, [piB1[...]], zeros16)
        plsc.store_scatter(h0, [Iaa], Aa)
        plsc.store_scatter(h0, [Iba], Ba)
        plsc.store_scatter(h1, [Iab], Ab)
        plsc.store_scatter(h1, [Ibb], Bb)
        piA0[...] = Iaa
        piB0[...] = Iba
        piA1[...] = Iab
        piB1[...] = Ibb
        pltpu.async_copy(h0, h_hbm.at[row0], sem_h)
        pltpu.async_copy(h1, h_hbm.at[row1], sem_h)

        @pl.when(r2 < RPW // 2 - 1)
        def _prefetch():
            _fetch_pre(row0 + 2, pre0)
            pltpu.async_copy(cmax_hbm.at[row0 + 2], cm0, sem_cm)
            _fetch_pre(row1 + 2, pre1)
            pltpu.async_copy(cmax_hbm.at[row1 + 2], cm1, sem_cm)
        return carry

    lax.fori_loop(0, RPW // 2, outer, jnp.int32(0))

    pltpu.make_async_copy(h0, h_hbm.at[base], sem_h).wait()
    pltpu.make_async_copy(h1, h_hbm.at[base], sem_h).wait()


def _sc_topk(pre, cmax):
    mesh = plsc.VectorSubcoreMesh(core_axis_name="c", subcore_axis_name="s")
    f = pl.kernel(
        _sc_body,
        out_type=jax.ShapeDtypeStruct((BATCH, N_FEATURES), jnp.float32),
        mesh=mesh,
        compiler_params=pltpu.CompilerParams(needs_layout_passes=False),
        scratch_types=[
            pltpu.VMEM((16, NCH + 1), jnp.float32),
            pltpu.VMEM((16, NCH + 1), jnp.float32),
            pltpu.VMEM((N_FEATURES,), jnp.float32),
            pltpu.VMEM((N_FEATURES,), jnp.float32),
            pltpu.VMEM((NCH,), jnp.float32),
            pltpu.VMEM((NCH,), jnp.float32),
            pltpu.VMEM((CAP,), jnp.int32),
            pltpu.VMEM((CAP,), jnp.int32),
            pltpu.VMEM((16,), jnp.int32),
            pltpu.VMEM((16,), jnp.int32),
            pltpu.VMEM((16,), jnp.int32),
            pltpu.VMEM((16,), jnp.int32),
            pltpu.SemaphoreType.DMA,
            pltpu.SemaphoreType.DMA,
            pltpu.SemaphoreType.DMA,
        ],
    )
    return f(pre, cmax)


def _decode_body(h_ref, w_ref, bdec_ref, out_ref):
    k = pl.program_id(1)
    acc = jnp.dot(h_ref[...], w_ref[...], preferred_element_type=jnp.float32)

    @pl.when(k == 0)
    def _init():
        out_ref[...] = acc + bdec_ref[...]

    @pl.when(k != 0)
    def _acc():
        out_ref[...] += acc


def _decode(h, W_dec, b_dec, bm=1024, bk=1024):
    m_blocks = BATCH // bm
    k_blocks = N_FEATURES // bk
    return pl.pallas_call(
        _decode_body,
        grid=(m_blocks, k_blocks),
        in_specs=[
            pl.BlockSpec((bm, bk), lambda m, k: (m, k)),
            pl.BlockSpec((bk, D_MODEL), lambda m, k: (k, 0)),
            pl.BlockSpec((1, D_MODEL), lambda m, k: (0, 0)),
        ],
        out_specs=pl.BlockSpec((bm, D_MODEL), lambda m, k: (m, 0)),
        out_shape=jax.ShapeDtypeStruct((BATCH, D_MODEL), jnp.float32),
    )(h, W_dec, b_dec[None, :])


def kernel(x, b_pre, W_enc, b_enc, W_dec, b_dec):
    pre, cmax = _encode(x, b_pre, W_enc, b_enc)
    h = _sc_topk(pre, cmax)
    x_hat = _decode(h, W_dec, b_dec)
    return (h, x_hat)


# speedup vs baseline: 1.3023x; 1.1559x over previous
import functools

import jax
import jax.numpy as jnp
from jax import lax
from jax.experimental import pallas as pl
from jax.experimental.pallas import tpu as pltpu
from jax.experimental.pallas import tpu_sc as plsc

BATCH = 4096
D_MODEL = 2048
N_FEATURES = 16384
K = 32

NC = 2
NS = 16
NW = NC * NS
RPW = BATCH // NW
NCH = 1024
NCV = NCH // 16
CAP = NCH + 16
CAP_E = 4096


def _encode_body(x_ref, bpre_ref, w_ref, benc_ref, pre_ref, cmax_ref):
    n = pl.program_id(1)
    xb = x_ref[...] - bpre_ref[...]
    acc = jnp.dot(xb, w_ref[...], preferred_element_type=jnp.float32)
    p = jnp.maximum(acc + benc_ref[...], 0.0)
    pre_ref[...] = p

    @pl.when(n % 16 == 0)
    def _init():
        cmax_ref[...] = p

    @pl.when(n % 16 != 0)
    def _acc():
        cmax_ref[...] = jnp.maximum(cmax_ref[...], p)


def _encode(x, b_pre, W_enc, b_enc, bm=1024, bn=512):
    m_blocks = BATCH // bm
    n_blocks = N_FEATURES // bn

    def c_of(n):
        return 2 * (n % 16) + n // 16

    return pl.pallas_call(
        _encode_body,
        grid=(m_blocks, n_blocks),
        in_specs=[
            pl.BlockSpec((bm, D_MODEL), lambda m, n: (m, 0)),
            pl.BlockSpec((1, D_MODEL), lambda m, n: (0, 0)),
            pl.BlockSpec((D_MODEL, bn), lambda m, n: (0, c_of(n))),
            pl.BlockSpec((1, bn), lambda m, n: (0, c_of(n))),
        ],
        out_specs=[
            pl.BlockSpec((bm, bn), lambda m, n: (m, c_of(n))),
            pl.BlockSpec((bm, bn), lambda m, n: (m, n // 16)),
        ],
        out_shape=[
            jax.ShapeDtypeStruct((BATCH, N_FEATURES), jnp.float32),
            jax.ShapeDtypeStruct((BATCH, NCH), jnp.float32),
        ],
    )(x, b_pre[None, :], W_enc, b_enc[None, :])


def _merge_chunk(Av, Ai, Bv, Bi, Cv, Ci):
    Cv, Ci = plsc.sort_key_val(Cv, Ci)
    Cr = lax.rev(Cv, (0,))
    Cir = lax.rev(Ci, (0,))
    m = Av >= Cr
    hv = jnp.where(m, Av, Cr)
    hi = jnp.where(m, Ai, Cir)
    lv = jnp.where(m, Cr, Av)
    li = jnp.where(m, Cir, Ai)
    Av, Ai = plsc.sort_key_val(hv, hi)
    lv, li = plsc.sort_key_val(lv, li)
    lr = lax.rev(lv, (0,))
    lir = lax.rev(li, (0,))
    m2 = Bv >= lr
    h2v = jnp.where(m2, Bv, lr)
    h2i = jnp.where(m2, Bi, lir)
    Bv, Bi = plsc.sort_key_val(h2v, h2i)
    return Av, Ai, Bv, Bi


_ABL_MERGE = 1


def _splat0(v):
    return lax.gather(
        v, jnp.zeros((16, 1), jnp.int32),
        lax.GatherDimensionNumbers(
            offset_dims=(), collapsed_slice_dims=(0,), start_index_map=(0,)),
        (1,), mode=lax.GatherScatterMode.PROMISE_IN_BOUNDS)


def _sc_body(pre_hbm, cmax_hbm, h_hbm,
             pre0, pre1, h0, h1, cm0, cm1, cids0, cids1,
             ev0, ev1, ei0, ei1,
             piA0, piB0, piA1, piB1, sem_pre, sem_cm, sem_h):
    wid = lax.axis_index("s") * NC + lax.axis_index("c")
    base = wid * RPW
    iota = lax.iota(jnp.int32, 16)
    zeros16 = jnp.zeros((16,), jnp.float32)
    neg1 = jnp.full((16,), -1.0, jnp.float32)

    def zstep(j, c):
        h0[pl.ds(j * 16, 16)] = zeros16
        h1[pl.ds(j * 16, 16)] = zeros16
        return c
    lax.fori_loop(0, N_FEATURES // 16, zstep, 0)

    def zstep2(j, c):
        ev0[pl.ds(j * 16, 16)] = zeros16
        ev1[pl.ds(j * 16, 16)] = zeros16
        ei0[pl.ds(j * 16, 16)] = iota * 0
        ei1[pl.ds(j * 16, 16)] = iota * 0
        return c
    lax.fori_loop(0, CAP_E // 16, zstep2, 0)
    piA0[...] = iota
    piB0[...] = iota + 16
    piA1[...] = iota
    piB1[...] = iota + 16

    def _fetch_pre(row, dst):
        for l in range(16):
            pltpu.async_copy(pre_hbm.at[row, pl.ds(l * NCH, NCH)],
                             dst.at[l, pl.ds(0, NCH)], sem_pre)

    _fetch_pre(base, pre0)
    pltpu.async_copy(cmax_hbm.at[base], cm0, sem_cm)
    _fetch_pre(base + 1, pre1)
    pltpu.async_copy(cmax_hbm.at[base + 1], cm1, sem_cm)

    def outer(r2, carry):
        row0 = base + 2 * r2
        row1 = row0 + 1
        for l in range(16):
            pltpu.make_async_copy(pre_hbm.at[row0, pl.ds(l * NCH, NCH)],
                                  pre0.at[l, pl.ds(0, NCH)], sem_pre).wait()
            pltpu.make_async_copy(pre_hbm.at[row1, pl.ds(l * NCH, NCH)],
                                  pre1.at[l, pl.ds(0, NCH)], sem_pre).wait()
        pltpu.make_async_copy(cmax_hbm.at[row0], cm0, sem_cm).wait()
        pltpu.make_async_copy(cmax_hbm.at[row1], cm1, sem_cm).wait()

        def t0step(j, LL):
            L1a, L2a, L1b, L2b = LL
            va = cm0[pl.ds(j * 16, 16)]
            vb = cm1[pl.ds(j * 16, 16)]
            ma = va > L1a
            mb = vb > L1b
            L2a = jnp.where(ma, L1a, jnp.maximum(L2a, va))
            L1a = jnp.where(ma, va, L1a)
            L2b = jnp.where(mb, L1b, jnp.maximum(L2b, vb))
            L1b = jnp.where(mb, vb, L1b)
            return (L1a, L2a, L1b, L2b)
        _, L2a, _, L2b = lax.fori_loop(0, NCV, t0step,
                                       (neg1, neg1, neg1, neg1))
        sL2a, _ = plsc.sort_key_val(L2a, iota)
        sL2b, _ = plsc.sort_key_val(L2b, iota)
        t0a = _splat0(sL2a)
        t0b = _splat0(sL2b)

        def cstep(j, cnts):
            cnta, cntb = cnts
            ids = iota + j * 16
            va = cm0[pl.ds(j * 16, 16)]
            vb = cm1[pl.ds(j * 16, 16)]
            ma = va >= t0a
            mb = vb >= t0b
            posa = cnta + plsc.cumsum(ma.astype(jnp.int32)) - 1
            posb = cntb + plsc.cumsum(mb.astype(jnp.int32)) - 1
            plsc.store_scatter(cids0, [posa], ids, mask=ma)
            plsc.store_scatter(cids1, [posb], ids, mask=mb)
            return (cnta + plsc.all_reduce_population_count(ma),
                    cntb + plsc.all_reduce_population_count(mb))
        cnta, cntb = lax.fori_loop(
            0, NCV, cstep,
            (jnp.zeros((16,), jnp.int32), jnp.zeros((16,), jnp.int32)))
        cntmax = jnp.maximum(cnta, cntb)

        def fcond(state):
            jv = state[0]
            return jnp.any(jv < cntmax * _ABL_MERGE)

        def fbody(state):
            jv, cea, ceb = state
            cida = plsc.load_gather(cids0, [jv])
            cidb = plsc.load_gather(cids1, [jv])
            cida = jnp.where(jv < cnta, cida, 0)
            cidb = jnp.where(jv < cntb, cidb, 0)
            va = plsc.load_gather(pre0, [iota, cida])
            vb = plsc.load_gather(pre1, [iota, cidb])
            fa = cida + NCH * iota
            fb = cidb + NCH * iota
            ma = (va >= t0a) & (jv < cnta)
            mb = (vb >= t0b) & (jv < cntb)
            posa = cea + plsc.cumsum(ma.astype(jnp.int32)) - 1
            posb = ceb + plsc.cumsum(mb.astype(jnp.int32)) - 1
            posa = jnp.minimum(posa, CAP_E - 1)
            posb = jnp.minimum(posb, CAP_E - 1)
            plsc.store_scatter(ev0, [posa], va, mask=ma)
            plsc.store_scatter(ei0, [posa], fa, mask=ma)
            plsc.store_scatter(ev1, [posb], vb, mask=mb)
            plsc.store_scatter(ei1, [posb], fb, mask=mb)
            return (jv + 1, cea + plsc.all_reduce_population_count(ma),
                    ceb + plsc.all_reduce_population_count(mb))
        z16i = jnp.zeros((16,), jnp.int32)
        _, cea, ceb = lax.while_loop(fcond, fbody, (z16i, z16i, z16i))
        cemax = jnp.maximum(cea, ceb)

        def mcond(state):
            jv = state[0]
            return jnp.any(jv * 16 < cemax)

        def mbody(state):
            jv, Aa, Iaa, Ba, Iba, Ab, Iab, Bb, Ibb = state
            pa = jv * 16 + iota
            pc = jnp.minimum(pa, CAP_E - 1)
            Cva = plsc.load_gather(ev0, [pc])
            Cia = plsc.load_gather(ei0, [pc])
            Cvb = plsc.load_gather(ev1, [pc])
            Cib = plsc.load_gather(ei1, [pc])
            Cva = jnp.where(pa < cea, Cva, neg1)
            Cvb = jnp.where(pa < ceb, Cvb, neg1)
            Aa, Iaa, Ba, Iba = _merge_chunk(Aa, Iaa, Ba, Iba, Cva, Cia)
            Ab, Iab, Bb, Ibb = _merge_chunk(Ab, Iab, Bb, Ibb, Cvb, Cib)
            return (jv + 1, Aa, Iaa, Ba, Iba, Ab, Iab, Bb, Ibb)
        init = (z16i, zeros16, iota, zeros16, iota + 16,
                zeros16, iota, zeros16, iota + 16)
        (_, Aa, Iaa, Ba, Iba, Ab, Iab, Bb, Ibb) = lax.while_loop(
            mcond, mbody, init)

        @pl.when(r2 > 0)
        def _wait_h():
            pltpu.make_async_copy(h0, h_hbm.at[row0], sem_h).wait()
            pltpu.make_async_copy(h1, h_hbm.at[row1], sem_h).wait()
        plsc.store_scatter(h0, [piA0[...]], zeros16)
        plsc.store_scatter(h0, [piB0[...]], zeros16)
        plsc.store_scatter(h1, [piA1[...]], zeros16)
        plsc.store_scatter(h1, [piB1[...]], zeros16)
        plsc.store_scatter(h0, [Iaa], Aa)
        plsc.store_scatter(h0, [Iba], Ba)
        plsc.store_scatter(h1, [Iab], Ab)
        plsc.store_scatter(h1, [Ibb], Bb)
        piA0[...] = Iaa
        piB0[...] = Iba
        piA1[...] = Iab
        piB1[...] = Ibb
        pltpu.async_copy(h0, h_hbm.at[row0], sem_h)
        pltpu.async_copy(h1, h_hbm.at[row1], sem_h)

        @pl.when(r2 < RPW // 2 - 1)
        def _prefetch():
            _fetch_pre(row0 + 2, pre0)
            pltpu.async_copy(cmax_hbm.at[row0 + 2], cm0, sem_cm)
            _fetch_pre(row1 + 2, pre1)
            pltpu.async_copy(cmax_hbm.at[row1 + 2], cm1, sem_cm)
        return carry

    lax.fori_loop(0, RPW // 2, outer, jnp.int32(0))

    pltpu.make_async_copy(h0, h_hbm.at[base], sem_h).wait()
    pltpu.make_async_copy(h1, h_hbm.at[base], sem_h).wait()


def _sc_topk(pre, cmax):
    mesh = plsc.VectorSubcoreMesh(core_axis_name="c", subcore_axis_name="s")
    f = pl.kernel(
        _sc_body,
        out_type=jax.ShapeDtypeStruct((BATCH, N_FEATURES), jnp.float32),
        mesh=mesh,
        compiler_params=pltpu.CompilerParams(needs_layout_passes=False),
        scratch_types=[
            pltpu.VMEM((16, NCH + 1), jnp.float32),
            pltpu.VMEM((16, NCH + 1), jnp.float32),
            pltpu.VMEM((N_FEATURES,), jnp.float32),
            pltpu.VMEM((N_FEATURES,), jnp.float32),
            pltpu.VMEM((NCH,), jnp.float32),
            pltpu.VMEM((NCH,), jnp.float32),
            pltpu.VMEM((CAP,), jnp.int32),
            pltpu.VMEM((CAP,), jnp.int32),
            pltpu.VMEM((CAP_E,), jnp.float32),
            pltpu.VMEM((CAP_E,), jnp.float32),
            pltpu.VMEM((CAP_E,), jnp.int32),
            pltpu.VMEM((CAP_E,), jnp.int32),
            pltpu.VMEM((16,), jnp.int32),
            pltpu.VMEM((16,), jnp.int32),
            pltpu.VMEM((16,), jnp.int32),
            pltpu.VMEM((16,), jnp.int32),
            pltpu.SemaphoreType.DMA,
            pltpu.SemaphoreType.DMA,
            pltpu.SemaphoreType.DMA,
        ],
    )
    return f(pre, cmax)


def _decode_body(h_ref, w_ref, bdec_ref, out_ref):
    k = pl.program_id(1)
    acc = jnp.dot(h_ref[...], w_ref[...], preferred_element_type=jnp.float32)

    @pl.when(k == 0)
    def _init():
        out_ref[...] = acc + bdec_ref[...]

    @pl.when(k != 0)
    def _acc():
        out_ref[...] += acc


def _decode(h, W_dec, b_dec, bm=1024, bk=1024):
    m_blocks = BATCH // bm
    k_blocks = N_FEATURES // bk
    return pl.pallas_call(
        _decode_body,
        grid=(m_blocks, k_blocks),
        in_specs=[
            pl.BlockSpec((bm, bk), lambda m, k: (m, k)),
            pl.BlockSpec((bk, D_MODEL), lambda m, k: (k, 0)),
            pl.BlockSpec((1, D_MODEL), lambda m, k: (0, 0)),
        ],
        out_specs=pl.BlockSpec((bm, D_MODEL), lambda m, k: (m, 0)),
        out_shape=jax.ShapeDtypeStruct((BATCH, D_MODEL), jnp.float32),
    )(h, W_dec, b_dec[None, :])


def kernel(x, b_pre, W_enc, b_enc, W_dec, b_dec):
    pre, cmax = _encode(x, b_pre, W_enc, b_enc)
    h = _sc_topk(pre, cmax)
    x_hat = _decode(h, W_dec, b_dec)
    return (h, x_hat)
